# baseline (speedup 1.0000x reference)
import functools

import jax
import jax.numpy as jnp
from jax import lax
from jax.experimental import pallas as pl
from jax.experimental.pallas import tpu as pltpu
from jax.experimental.pallas import tpu_sc as plsc

_W_CHAMFER_OPP = 0.55
_W_LAPLACE = 0.5
_W_MOVE = 0.033
_W_EDGE = 0.1
_W_NORMAL = 0.00016
_LAP_CONST = (0.2, 1.0, 1.0)

_B = 4
_NGT = 4096
_RG = 512

_NC = 2
_NS = 16
_NW = _NC * _NS
_TPB = _NW // _B


def _round_up(x, m):
    return (x + m - 1) // m * m


_IDXM = 4095


def _chamfer_body(segs, gt_ref, predT_ref, d1_ref, d2_ref, idx2_ref,
                  runkey_ref):
    b = pl.program_id(0)
    c = pl.program_id(1)
    ncols = predT_ref.shape[2]

    g = gt_ref[0]
    gx = g[:, 0:1]
    gy = g[:, 1:2]
    gz = g[:, 2:3]
    px = predT_ref[0, 0:1, :]
    py = predT_ref[0, 1:2, :]
    pz = predT_ref[0, 2:3, :]

    lane = lax.broadcasted_iota(jnp.int32, (1, ncols), 1)
    valid = lane < -1
    for off, width, nreal in segs:
        valid = jnp.logical_or(
            valid, jnp.logical_and(lane >= off, lane < off + nreal))
    big = jnp.float32(3.0e38)
    p2 = jnp.where(valid, px * px + py * py + pz * pz, big)

    g2 = gx * gx + gy * gy + gz * gz
    gx2 = -2.0 * gx
    gy2 = -2.0 * gy
    gz2 = -2.0 * gz
    m = gx2 * px + gy2 * py + gz2 * pz
    d = jnp.maximum((g2 + m) + p2, 0.0)

    rows = lax.broadcasted_iota(jnp.int32, d.shape, 0) + c * _RG
    key = lax.bitcast_convert_type(
        (lax.bitcast_convert_type(d, jnp.int32) & jnp.int32(~_IDXM)) | rows,
        jnp.float32)

    @pl.when(c == 0)
    def _init_scratch():
        runkey_ref[...] = jnp.full(runkey_ref.shape, jnp.float32(3.3e38),
                                   jnp.float32)

    @pl.when(jnp.logical_and(b == 0, c == 0))
    def _init_outs():
        for i in range(len(segs)):
            d1_ref[i][...] = jnp.zeros((1, 1), jnp.float32)
            d2_ref[i][...] = jnp.zeros((1, 1), jnp.float32)

    for i, (off, width, nreal) in enumerate(segs):
        seg_key = key[:, off:off + width]
        row_min = jnp.min(seg_key, axis=1, keepdims=True)
        part1 = jnp.sum(lax.bitcast_convert_type(
            lax.bitcast_convert_type(row_min, jnp.int32) & jnp.int32(~_IDXM),
            jnp.float32), axis=0, keepdims=True)
        d1_ref[i][...] += part1

    col_min = jnp.min(key, axis=0, keepdims=True)
    new_key = jnp.minimum(col_min, runkey_ref[...])
    runkey_ref[...] = new_key

    @pl.when(c == pl.num_programs(1) - 1)
    def _finish():
        bits = lax.bitcast_convert_type(new_key, jnp.int32)
        vals = lax.bitcast_convert_type(bits & jnp.int32(~_IDXM),
                                        jnp.float32)
        masked = jnp.where(valid, vals, 0.0)
        for i, (off, width, nreal) in enumerate(segs):
            d2_ref[i][...] += jnp.sum(masked[:, off:off + width],
                                      axis=1, keepdims=True)
        idx2_ref[0] = bits & _IDXM


def _chamfer_all(gt, predT_cat, segs):
    w = predT_cat.shape[2]
    nchunks = _NGT // _RG
    grid = (_B, nchunks)
    nseg = len(segs)

    def body(gt_ref, predT_ref, *outs):
        d1 = outs[:nseg]
        d2 = outs[nseg:2 * nseg]
        idx2_ref = outs[2 * nseg]
        runkey_ref = outs[2 * nseg + 1]
        _chamfer_body(segs, gt_ref, predT_ref, d1, d2, idx2_ref, runkey_ref)

    scalar_spec = pl.BlockSpec((1, 1), lambda b, c: (0, 0))
    scalar_shape = jax.ShapeDtypeStruct((1, 1), jnp.float32)
    return pl.pallas_call(
        body,
        grid=grid,
        in_specs=[
            pl.BlockSpec((1, _RG, 3), lambda b, c: (b, c, 0)),
            pl.BlockSpec((1, 3, w), lambda b, c: (b, 0, 0)),
        ],
        out_specs=([scalar_spec] * (2 * nseg)
                   + [pl.BlockSpec((1, 1, w), lambda b, c: (b, 0, 0))]),
        out_shape=([scalar_shape] * (2 * nseg)
                   + [jax.ShapeDtypeStruct((_B, 1, w), jnp.int32)]),
        scratch_shapes=[
            pltpu.VMEM((1, w), jnp.float32),
        ],
    )(gt, predT_cat)


def _rsqrt_nr(x):
    xi = plsc.bitcast(x, jnp.int32)
    yi = jnp.int32(0x5F3759DF) - (xi >> 1)
    y = plsc.bitcast(yi, jnp.float32)
    for _ in range(3):
        y = y * (1.5 - 0.5 * x * y * y)
    return y


def _sc_mesh():
    return plsc.VectorSubcoreMesh(core_axis_name="c", subcore_axis_name="s",
                                  num_cores=_NC, num_subcores=_NS)


def _sc_wid():
    cid = lax.axis_index("c")
    sid = lax.axis_index("s")
    wid = sid * _NC + cid
    return wid // _TPB, wid % _TPB, wid


def _sc_body(pps, eps, offs, w, *refs):
    nlev = len(pps)
    pcat, bcat, icat, nrm_h = refs[0], refs[1], refs[2], refs[3]
    ins = refs[4: 4 + 2 * nlev]
    out_hbm = refs[4 + 2 * nlev]
    scr = refs[5 + 2 * nlev:]
    nv = scr[6 * nlev]
    ov = scr[6 * nlev + 1]

    b, t, wid = _sc_wid()
    zero = jnp.zeros((16,), jnp.float32)

    pltpu.sync_copy(nrm_h.at[b], nv)

    for lvl in range(nlev):
        eh, lh = ins[2 * lvl: 2 * lvl + 2]
        pv, bv, dv, lv, ev, iv = scr[6 * lvl: 6 * lvl + 6]
        pp = pps[lvl]
        ep = eps[lvl]
        off = offs[lvl]

        for c in range(3):
            pltpu.sync_copy(pcat.at[b, pl.ds(c * w + off, pp)],
                            pv.at[pl.ds(c * pp, pp)])
            pltpu.sync_copy(bcat.at[b, pl.ds(c * w + off, pp)],
                            bv.at[pl.ds(c * pp, pp)])
        pltpu.sync_copy(icat.at[b, pl.ds(off, pp)], iv)
        pltpu.sync_copy(eh, ev)
        pltpu.sync_copy(lh, lv)

        def dbody(k, carry):
            s = pl.ds(k * 16, 16)
            dv[s] = bv[s] - pv[s]
            return carry
        lax.fori_loop(0, 3 * pp // 16, dbody, 0)

        nech = ep // (16 * _TPB)

        def ebody(k, accs):
            e_acc, c_acc = accs
            j = t * nech + k
            ea = ev[pl.ds(j * 16, 16)]
            eb = ev[pl.ds(ep + j * 16, 16)]
            dd = zero
            diffs = []
            for c in range(3):
                va = plsc.load_gather(pv, [ea + c * pp])
                vb = plsc.load_gather(pv, [eb + c * pp])
                df = va - vb
                diffs.append(df)
                dd = dd + df * df
            e_acc = e_acc + dd
            gi3 = plsc.load_gather(iv, [ea]) * 3
            nn2 = zero
            dot = zero
            for c in range(3):
                nc = plsc.load_gather(nv, [gi3 + c])
                nn2 = nn2 + nc * nc
                dot = dot + diffs[c] * nc
            cos = (jnp.abs(dot)
                   * _rsqrt_nr(jnp.maximum(dd, 1e-24))
                   * _rsqrt_nr(jnp.maximum(nn2, 1e-24)))
            return (e_acc, c_acc + cos)

        e_acc, c_acc = lax.fori_loop(0, nech, ebody, (zero, zero))

        nvch = pp // (16 * _TPB)

        def vbody(k, accs):
            l_acc, m_acc = accs
            j = t * nvch + k
            dcs = [dv[pl.ds(c * pp + j * 16, 16)] for c in range(3)]
            mm = dcs[0] * dcs[0] + dcs[1] * dcs[1] + dcs[2] * dcs[2]
            ns = [zero, zero, zero]
            for nb in range(8):
                idx = lv[pl.ds(nb * pp + j * 16, 16)]
                for c in range(3):
                    ns[c] = ns[c] + plsc.load_gather(dv, [idx + c * pp])
            cnt = lv[pl.ds(9 * pp + j * 16, 16)].astype(jnp.float32)
            ll = zero
            for c in range(3):
                lc = dcs[c] - ns[c] / cnt
                ll = ll + lc * lc
            return (l_acc + ll, m_acc + mm)

        l_acc, m_acc = lax.fori_loop(0, nvch, vbody, (zero, zero))

        ov[pl.ds((lvl * 4 + 0) * 16, 16)] = e_acc
        ov[pl.ds((lvl * 4 + 1) * 16, 16)] = c_acc
        ov[pl.ds((lvl * 4 + 2) * 16, 16)] = l_acc
        ov[pl.ds((lvl * 4 + 3) * 16, 16)] = m_acc

    pltpu.sync_copy(ov, out_hbm.at[wid])


def _sc_regularizers(predT_cat, beforeT_cat, idx2_all, gt_normals,
                     edge_flats, lap_flats, offs, pps, eps):
    nlev = len(pps)
    w = predT_cat.shape[2]
    scratch = []
    for i in range(nlev):
        scratch += [
            pltpu.VMEM((3 * pps[i],), jnp.float32),
            pltpu.VMEM((3 * pps[i],), jnp.float32),
            pltpu.VMEM((3 * pps[i],), jnp.float32),
            pltpu.VMEM((10 * pps[i],), jnp.int32),
            pltpu.VMEM((2 * eps[i],), jnp.int32),
            pltpu.VMEM((pps[i],), jnp.int32),
        ]
    scratch += [
        pltpu.VMEM((3 * _NGT,), jnp.float32),
        pltpu.VMEM((nlev * 4 * 16,), jnp.float32),
    ]
    fn = pl.kernel(
        functools.partial(_sc_body, tuple(pps), tuple(eps), tuple(offs), w),
        out_type=jax.ShapeDtypeStruct((_NW, nlev * 4 * 16), jnp.float32),
        mesh=_sc_mesh(),
        scratch_types=scratch,
        compiler_params=pltpu.CompilerParams(needs_layout_passes=False),
    )
    args = [predT_cat.reshape(_B, 3 * w), beforeT_cat.reshape(_B, 3 * w),
            idx2_all.reshape(_B, w), gt_normals.reshape(_B, 3 * _NGT)]
    for i in range(nlev):
        args += [edge_flats[i], lap_flats[i]]
    return fn(*args).reshape(_NW, nlev, 4, 16)


def kernel(pred0, pred1, pred2, pred_before0, pred_before1, pred_before2,
           gt_points, gt_normals,
           edges0, edges1, edges2, lap_idx0, lap_idx1, lap_idx2):
    preds = [pred0, pred1, pred2]
    befores = [pred_before0, pred_before1, pred_before2]
    edges = [edges0, edges1, edges2]
    laps = [lap_idx0, lap_idx1, lap_idx2]

    psizes, nes, pps, eps = [], [], [], []
    pred_parts, before_parts, edge_flats, lap_flats = [], [], [], []
    for i in range(3):
        p = preds[i].shape[1]
        ne = edges[i].shape[0]
        pp = _round_up(p, 128)
        ep = _round_up(ne, 128)
        psizes.append(p)
        nes.append(ne)
        pps.append(pp)
        eps.append(ep)
        zpad = jnp.zeros((_B, pp - p, 3), jnp.float32)
        pred_parts += [preds[i], zpad]
        before_parts += [befores[i], zpad]
        edge_flats.append(
            jnp.pad(edges[i], ((0, ep - ne), (0, 0))).T.reshape(-1))
        lapT = jnp.pad(laps[i], ((0, pp - p), (0, 0)), constant_values=p).T
        lap_flats.append(lapT.reshape(-1))

    offs = [0, pps[0], pps[0] + pps[1]]
    segs = tuple((offs[i], pps[i], psizes[i]) for i in range(3))
    predT_cat = jnp.transpose(jnp.concatenate(pred_parts, axis=1), (0, 2, 1))
    beforeT_cat = jnp.transpose(jnp.concatenate(before_parts, axis=1),
                                (0, 2, 1))
    outs = _chamfer_all(gt_points, predT_cat, segs)
    d1s, d2s, idx2_all = outs[:3], outs[3:6], outs[6]
    chamfers = [d1s[i][0, 0] / (_B * _NGT)
                + _W_CHAMFER_OPP * d2s[i][0, 0] / (_B * psizes[i])
                for i in range(3)]

    parts = _sc_regularizers(predT_cat, beforeT_cat, idx2_all, gt_normals,
                             edge_flats, lap_flats, offs, pps, eps)
    sums = jnp.sum(parts, axis=(0, 3))

    loss = jnp.float32(0.0)
    for i in range(3):
        p, ne = psizes[i], nes[i]
        loss = loss + chamfers[i]
        loss = loss + _W_NORMAL * sums[i, 1] / (_B * ne)
        loss = loss + _W_EDGE * sums[i, 0] / (_B * ne)
        loss = loss + _W_LAPLACE * _LAP_CONST[i] * sums[i, 2] / (_B * p)
        if i > 0:
            loss = loss + _W_MOVE * _LAP_CONST[i] * sums[i, 3] / (_B * p)
    return loss


# speedup vs baseline: 1.1400x; 1.1400x over previous
import functools

import jax
import jax.numpy as jnp
from jax import lax
from jax.experimental import pallas as pl
from jax.experimental.pallas import tpu as pltpu
from jax.experimental.pallas import tpu_sc as plsc

_W_CHAMFER_OPP = 0.55
_W_LAPLACE = 0.5
_W_MOVE = 0.033
_W_EDGE = 0.1
_W_NORMAL = 0.00016
_LAP_CONST = (0.2, 1.0, 1.0)

_B = 4
_NGT = 4096
_RG = 512

_NC = 2
_NS = 16
_NW = _NC * _NS
_TPB = _NW // _B


def _round_up(x, m):
    return (x + m - 1) // m * m


_IDXM = 4095


def _chamfer_body(segs, gt_ref, predT_ref, d1_ref, d2_ref, idx2_ref,
                  runkey_ref):
    b = pl.program_id(0)
    c = pl.program_id(1)
    ncols = predT_ref.shape[2]

    g = gt_ref[0]
    gx = g[:, 0:1]
    gy = g[:, 1:2]
    gz = g[:, 2:3]
    px = predT_ref[0, 0:1, :]
    py = predT_ref[0, 1:2, :]
    pz = predT_ref[0, 2:3, :]

    lane = lax.broadcasted_iota(jnp.int32, (1, ncols), 1)
    valid = lane < -1
    for off, width, nreal in segs:
        valid = jnp.logical_or(
            valid, jnp.logical_and(lane >= off, lane < off + nreal))
    big = jnp.float32(3.0e38)
    p2 = jnp.where(valid, px * px + py * py + pz * pz, big)

    g2 = gx * gx + gy * gy + gz * gz
    gx2 = -2.0 * gx
    gy2 = -2.0 * gy
    gz2 = -2.0 * gz
    m = gx2 * px + gy2 * py + gz2 * pz
    d = (g2 + m) + p2

    rows = lax.broadcasted_iota(jnp.int32, d.shape, 0) + c * _RG
    key = lax.bitcast_convert_type(
        (lax.bitcast_convert_type(d, jnp.int32) & jnp.int32(~_IDXM)) | rows,
        jnp.float32)

    @pl.when(c == 0)
    def _init_scratch():
        runkey_ref[...] = jnp.full(runkey_ref.shape, jnp.float32(3.3e38),
                                   jnp.float32)

    @pl.when(jnp.logical_and(b == 0, c == 0))
    def _init_outs():
        for i in range(len(segs)):
            d1_ref[i][...] = jnp.zeros((1, 1), jnp.float32)
            d2_ref[i][...] = jnp.zeros((1, 1), jnp.float32)

    for i, (off, width, nreal) in enumerate(segs):
        seg_key = key[:, off:off + width]
        row_min = jnp.min(seg_key, axis=1, keepdims=True)
        part1 = jnp.sum(lax.bitcast_convert_type(
            lax.bitcast_convert_type(row_min, jnp.int32) & jnp.int32(~_IDXM),
            jnp.float32), axis=0, keepdims=True)
        d1_ref[i][...] += part1

    col_min = jnp.min(key, axis=0, keepdims=True)
    new_key = jnp.minimum(col_min, runkey_ref[...])
    runkey_ref[...] = new_key

    @pl.when(c == pl.num_programs(1) - 1)
    def _finish():
        bits = lax.bitcast_convert_type(new_key, jnp.int32)
        vals = lax.bitcast_convert_type(bits & jnp.int32(~_IDXM),
                                        jnp.float32)
        masked = jnp.where(valid, vals, 0.0)
        for i, (off, width, nreal) in enumerate(segs):
            d2_ref[i][...] += jnp.sum(masked[:, off:off + width],
                                      axis=1, keepdims=True)
        idx2_ref[0] = bits & _IDXM


def _chamfer_all(gt, predT_cat, segs):
    w = predT_cat.shape[2]
    nchunks = _NGT // _RG
    grid = (_B, nchunks)
    nseg = len(segs)

    def body(gt_ref, predT_ref, *outs):
        d1 = outs[:nseg]
        d2 = outs[nseg:2 * nseg]
        idx2_ref = outs[2 * nseg]
        runkey_ref = outs[2 * nseg + 1]
        _chamfer_body(segs, gt_ref, predT_ref, d1, d2, idx2_ref, runkey_ref)

    scalar_spec = pl.BlockSpec((1, 1), lambda b, c: (0, 0))
    scalar_shape = jax.ShapeDtypeStruct((1, 1), jnp.float32)
    return pl.pallas_call(
        body,
        grid=grid,
        in_specs=[
            pl.BlockSpec((1, _RG, 3), lambda b, c: (b, c, 0)),
            pl.BlockSpec((1, 3, w), lambda b, c: (b, 0, 0)),
        ],
        out_specs=([scalar_spec] * (2 * nseg)
                   + [pl.BlockSpec((1, 1, w), lambda b, c: (b, 0, 0))]),
        out_shape=([scalar_shape] * (2 * nseg)
                   + [jax.ShapeDtypeStruct((_B, 1, w), jnp.int32)]),
        scratch_shapes=[
            pltpu.VMEM((1, w), jnp.float32),
        ],
    )(gt, predT_cat)


def _rsqrt_nr(x):
    xi = plsc.bitcast(x, jnp.int32)
    yi = jnp.int32(0x5F3759DF) - (xi >> 1)
    y = plsc.bitcast(yi, jnp.float32)
    for _ in range(3):
        y = y * (1.5 - 0.5 * x * y * y)
    return y


def _sc_mesh():
    return plsc.VectorSubcoreMesh(core_axis_name="c", subcore_axis_name="s",
                                  num_cores=_NC, num_subcores=_NS)


def _sc_wid():
    cid = lax.axis_index("c")
    sid = lax.axis_index("s")
    wid = sid * _NC + cid
    return wid // _TPB, wid % _TPB, wid


def _sc_body(pps, eps, *refs):
    nlev = len(pps)
    ins = refs[: 5 * nlev + 1]
    out_hbm = refs[5 * nlev + 1]
    scr = refs[5 * nlev + 2:]
    nrm_h = ins[5 * nlev]
    nv = scr[6 * nlev]
    ov = scr[6 * nlev + 1]

    b, t, wid = _sc_wid()
    zero = jnp.zeros((16,), jnp.float32)

    pltpu.sync_copy(nrm_h.at[b], nv)

    for lvl in range(nlev):
        ph, bh, eh, lh, ih = ins[5 * lvl: 5 * lvl + 5]
        pv, bv, dv, lv, ev, iv = scr[6 * lvl: 6 * lvl + 6]
        pp = pps[lvl]
        ep = eps[lvl]

        pltpu.sync_copy(ph.at[b], pv)
        pltpu.sync_copy(bh.at[b], bv)
        pltpu.sync_copy(eh, ev)
        pltpu.sync_copy(lh, lv)
        pltpu.sync_copy(ih.at[b], iv)

        def dbody(k, carry):
            s = pl.ds(k * 16, 16)
            dv[s] = bv[s] - pv[s]
            return carry
        lax.fori_loop(0, 3 * pp // 16, dbody, 0)

        nech = ep // (16 * _TPB)

        def ebody(k, accs):
            e_acc, c_acc = accs
            j = t * nech + k
            ea = ev[pl.ds(j * 16, 16)]
            eb = ev[pl.ds(ep + j * 16, 16)]
            dd = zero
            diffs = []
            for c in range(3):
                va = plsc.load_gather(pv, [ea + c * pp])
                vb = plsc.load_gather(pv, [eb + c * pp])
                df = va - vb
                diffs.append(df)
                dd = dd + df * df
            e_acc = e_acc + dd
            gi = plsc.load_gather(iv, [ea])
            nn2 = zero
            dot = zero
            for c in range(3):
                nc = plsc.load_gather(nv, [gi + c * _NGT])
                nn2 = nn2 + nc * nc
                dot = dot + diffs[c] * nc
            cos = (jnp.abs(dot)
                   * _rsqrt_nr(jnp.maximum(dd, 1e-24))
                   * _rsqrt_nr(jnp.maximum(nn2, 1e-24)))
            return (e_acc, c_acc + cos)

        e_acc, c_acc = lax.fori_loop(0, nech, ebody, (zero, zero))

        nvch = pp // (16 * _TPB)

        def vbody(k, accs):
            l_acc, m_acc = accs
            j = t * nvch + k
            dcs = [dv[pl.ds(c * pp + j * 16, 16)] for c in range(3)]
            mm = dcs[0] * dcs[0] + dcs[1] * dcs[1] + dcs[2] * dcs[2]
            ns = [zero, zero, zero]
            for nb in range(8):
                idx = lv[pl.ds(nb * pp + j * 16, 16)]
                for c in range(3):
                    ns[c] = ns[c] + plsc.load_gather(dv, [idx + c * pp])
            cnt = lv[pl.ds(9 * pp + j * 16, 16)].astype(jnp.float32)
            ll = zero
            for c in range(3):
                lc = dcs[c] - ns[c] / cnt
                ll = ll + lc * lc
            return (l_acc + ll, m_acc + mm)

        l_acc, m_acc = lax.fori_loop(0, nvch, vbody, (zero, zero))

        ov[pl.ds((lvl * 4 + 0) * 16, 16)] = e_acc
        ov[pl.ds((lvl * 4 + 1) * 16, 16)] = c_acc
        ov[pl.ds((lvl * 4 + 2) * 16, 16)] = l_acc
        ov[pl.ds((lvl * 4 + 3) * 16, 16)] = m_acc

    pltpu.sync_copy(ov, out_hbm.at[wid])


def _sc_regularizers(predTs, beforeTs, edge_flats, lap_flats, idx2s,
                     normalsT, pps, eps):
    nlev = len(pps)
    scratch = []
    for i in range(nlev):
        scratch += [
            pltpu.VMEM((3 * pps[i],), jnp.float32),
            pltpu.VMEM((3 * pps[i],), jnp.float32),
            pltpu.VMEM((3 * pps[i],), jnp.float32),
            pltpu.VMEM((10 * pps[i],), jnp.int32),
            pltpu.VMEM((2 * eps[i],), jnp.int32),
            pltpu.VMEM((pps[i],), jnp.int32),
        ]
    scratch += [
        pltpu.VMEM((3 * _NGT,), jnp.float32),
        pltpu.VMEM((nlev * 4 * 16,), jnp.float32),
    ]
    fn = pl.kernel(
        functools.partial(_sc_body, tuple(pps), tuple(eps)),
        out_type=jax.ShapeDtypeStruct((_NW, nlev * 4 * 16), jnp.float32),
        mesh=_sc_mesh(),
        scratch_types=scratch,
        compiler_params=pltpu.CompilerParams(needs_layout_passes=False),
    )
    args = []
    for i in range(nlev):
        args += [predTs[i].reshape(_B, 3 * pps[i]),
                 beforeTs[i].reshape(_B, 3 * pps[i]),
                 edge_flats[i], lap_flats[i], idx2s[i]]
    args.append(normalsT.reshape(_B, 3 * _NGT))
    return fn(*args).reshape(_NW, nlev, 4, 16)


def kernel(pred0, pred1, pred2, pred_before0, pred_before1, pred_before2,
           gt_points, gt_normals,
           edges0, edges1, edges2, lap_idx0, lap_idx1, lap_idx2):
    preds = [pred0, pred1, pred2]
    befores = [pred_before0, pred_before1, pred_before2]
    edges = [edges0, edges1, edges2]
    laps = [lap_idx0, lap_idx1, lap_idx2]

    psizes, nes, pps, eps = [], [], [], []
    predTs, beforeTs, edge_flats, lap_flats = [], [], [], []
    for i in range(3):
        p = preds[i].shape[1]
        ne = edges[i].shape[0]
        pp = _round_up(p, 128)
        ep = _round_up(ne, 128)
        psizes.append(p)
        nes.append(ne)
        pps.append(pp)
        eps.append(ep)
        predTs.append(jnp.pad(jnp.transpose(preds[i], (0, 2, 1)),
                              ((0, 0), (0, 0), (0, pp - p))))
        beforeTs.append(jnp.pad(jnp.transpose(befores[i], (0, 2, 1)),
                                ((0, 0), (0, 0), (0, pp - p))))
        edge_flats.append(
            jnp.pad(edges[i], ((0, ep - ne), (0, 0))).T.reshape(-1))
        lapT = jnp.pad(laps[i], ((0, pp - p), (0, 0)), constant_values=p).T
        lapT = lapT.at[9, p:].set(1)
        lap_flats.append(lapT.reshape(-1))

    offs = [0, pps[0], pps[0] + pps[1]]
    segs = tuple((offs[i], pps[i], psizes[i]) for i in range(3))
    predT_cat = jnp.concatenate(predTs, axis=2)
    outs = _chamfer_all(gt_points, predT_cat, segs)
    d1s, d2s, idx2_all = outs[:3], outs[3:6], outs[6]
    idx2s = [idx2_all[:, 0, offs[i]:offs[i] + pps[i]] for i in range(3)]
    chamfers = [d1s[i][0, 0] / (_B * _NGT)
                + _W_CHAMFER_OPP * d2s[i][0, 0] / (_B * psizes[i])
                for i in range(3)]

    normalsT = jnp.transpose(gt_normals, (0, 2, 1))
    parts = _sc_regularizers(predTs, beforeTs, edge_flats, lap_flats,
                             idx2s, normalsT, pps, eps)
    sums = jnp.sum(parts, axis=(0, 3))

    loss = jnp.float32(0.0)
    for i in range(3):
        p, ne = psizes[i], nes[i]
        loss = loss + chamfers[i]
        loss = loss + _W_NORMAL * sums[i, 1] / (_B * ne)
        loss = loss + _W_EDGE * sums[i, 0] / (_B * ne)
        loss = loss + _W_LAPLACE * _LAP_CONST[i] * sums[i, 2] / (_B * p)
        if i > 0:
            loss = loss + _W_MOVE * _LAP_CONST[i] * sums[i, 3] / (_B * p)
    return loss


# speedup vs baseline: 1.1410x; 1.0008x over previous
import functools

import jax
import jax.numpy as jnp
from jax import lax
from jax.experimental import pallas as pl
from jax.experimental.pallas import tpu as pltpu
from jax.experimental.pallas import tpu_sc as plsc

_W_CHAMFER_OPP = 0.55
_W_LAPLACE = 0.5
_W_MOVE = 0.033
_W_EDGE = 0.1
_W_NORMAL = 0.00016
_LAP_CONST = (0.2, 1.0, 1.0)

_B = 4
_NGT = 4096
_RG = 512

_NC = 2
_NS = 16
_NW = _NC * _NS
_TPB = _NW // _B


def _round_up(x, m):
    return (x + m - 1) // m * m


_IDXM = 4095


def _chamfer_body(segs, gt_ref, predT_ref, d1_ref, d2_ref, idx2_ref,
                  runkey_ref):
    b = pl.program_id(0)
    c = pl.program_id(1)
    ncols = predT_ref.shape[2]

    g = gt_ref[0]
    gx = g[:, 0:1]
    gy = g[:, 1:2]
    gz = g[:, 2:3]
    px = predT_ref[0, 0:1, :]
    py = predT_ref[0, 1:2, :]
    pz = predT_ref[0, 2:3, :]

    lane = lax.broadcasted_iota(jnp.int32, (1, ncols), 1)
    valid = lane < -1
    for off, width, nreal in segs:
        valid = jnp.logical_or(
            valid, jnp.logical_and(lane >= off, lane < off + nreal))
    big = jnp.float32(3.0e38)
    p2 = jnp.where(valid, px * px + py * py + pz * pz, big)

    g2 = gx * gx + gy * gy + gz * gz
    gx2 = -2.0 * gx
    gy2 = -2.0 * gy
    gz2 = -2.0 * gz
    m = gx2 * px + gy2 * py + gz2 * pz
    d = (g2 + m) + p2

    rows = lax.broadcasted_iota(jnp.int32, d.shape, 0)
    key = lax.bitcast_convert_type(
        (lax.bitcast_convert_type(d, jnp.int32) & jnp.int32(~_IDXM)) | rows,
        jnp.float32)

    @pl.when(c == 0)
    def _init_scratch():
        runkey_ref[...] = jnp.full(runkey_ref.shape, jnp.float32(3.3e38),
                                   jnp.float32)

    @pl.when(jnp.logical_and(b == 0, c == 0))
    def _init_outs():
        for i in range(len(segs)):
            d1_ref[i][...] = jnp.zeros((1, 1), jnp.float32)
            d2_ref[i][...] = jnp.zeros((1, 1), jnp.float32)

    for i, (off, width, nreal) in enumerate(segs):
        seg_key = key[:, off:off + width]
        row_min = jnp.min(seg_key, axis=1, keepdims=True)
        part1 = jnp.sum(lax.bitcast_convert_type(
            lax.bitcast_convert_type(row_min, jnp.int32) & jnp.int32(~_IDXM),
            jnp.float32), axis=0, keepdims=True)
        d1_ref[i][...] += part1

    col_min = jnp.min(key, axis=0, keepdims=True)
    col_glob = lax.bitcast_convert_type(
        lax.bitcast_convert_type(col_min, jnp.int32) | (c << 9), jnp.float32)
    new_key = jnp.minimum(col_glob, runkey_ref[...])
    runkey_ref[...] = new_key

    @pl.when(c == pl.num_programs(1) - 1)
    def _finish():
        bits = lax.bitcast_convert_type(new_key, jnp.int32)
        vals = lax.bitcast_convert_type(bits & jnp.int32(~_IDXM),
                                        jnp.float32)
        masked = jnp.where(valid, vals, 0.0)
        for i, (off, width, nreal) in enumerate(segs):
            d2_ref[i][...] += jnp.sum(masked[:, off:off + width],
                                      axis=1, keepdims=True)
        idx2_ref[0] = bits & _IDXM


def _chamfer_all(gt, predT_cat, segs):
    w = predT_cat.shape[2]
    nchunks = _NGT // _RG
    grid = (_B, nchunks)
    nseg = len(segs)

    def body(gt_ref, predT_ref, *outs):
        d1 = outs[:nseg]
        d2 = outs[nseg:2 * nseg]
        idx2_ref = outs[2 * nseg]
        runkey_ref = outs[2 * nseg + 1]
        _chamfer_body(segs, gt_ref, predT_ref, d1, d2, idx2_ref, runkey_ref)

    scalar_spec = pl.BlockSpec((1, 1), lambda b, c: (0, 0))
    scalar_shape = jax.ShapeDtypeStruct((1, 1), jnp.float32)
    return pl.pallas_call(
        body,
        grid=grid,
        in_specs=[
            pl.BlockSpec((1, _RG, 3), lambda b, c: (b, c, 0)),
            pl.BlockSpec((1, 3, w), lambda b, c: (b, 0, 0)),
        ],
        out_specs=([scalar_spec] * (2 * nseg)
                   + [pl.BlockSpec((1, 1, w), lambda b, c: (b, 0, 0))]),
        out_shape=([scalar_shape] * (2 * nseg)
                   + [jax.ShapeDtypeStruct((_B, 1, w), jnp.int32)]),
        scratch_shapes=[
            pltpu.VMEM((1, w), jnp.float32),
        ],
    )(gt, predT_cat)


def _rsqrt_nr(x):
    xi = plsc.bitcast(x, jnp.int32)
    yi = jnp.int32(0x5F3759DF) - (xi >> 1)
    y = plsc.bitcast(yi, jnp.float32)
    for _ in range(3):
        y = y * (1.5 - 0.5 * x * y * y)
    return y


def _sc_mesh():
    return plsc.VectorSubcoreMesh(core_axis_name="c", subcore_axis_name="s",
                                  num_cores=_NC, num_subcores=_NS)


def _sc_wid():
    cid = lax.axis_index("c")
    sid = lax.axis_index("s")
    wid = sid * _NC + cid
    return wid // _TPB, wid % _TPB, wid


def _sc_body(pps, eps, *refs):
    nlev = len(pps)
    ins = refs[: 5 * nlev + 1]
    out_hbm = refs[5 * nlev + 1]
    scr = refs[5 * nlev + 2:]
    nrm_h = ins[5 * nlev]
    nv = scr[6 * nlev]
    ov = scr[6 * nlev + 1]

    b, t, wid = _sc_wid()
    zero = jnp.zeros((16,), jnp.float32)

    pltpu.sync_copy(nrm_h.at[b], nv)

    for lvl in range(nlev):
        ph, bh, eh, lh, ih = ins[5 * lvl: 5 * lvl + 5]
        pv, bv, dv, lv, ev, iv = scr[6 * lvl: 6 * lvl + 6]
        pp = pps[lvl]
        ep = eps[lvl]

        pltpu.sync_copy(ph.at[b], pv)
        pltpu.sync_copy(bh.at[b], bv)
        pltpu.sync_copy(eh, ev)
        pltpu.sync_copy(lh, lv)
        pltpu.sync_copy(ih.at[b], iv)

        def dbody(k, carry):
            s = pl.ds(k * 16, 16)
            dv[s] = bv[s] - pv[s]
            return carry
        lax.fori_loop(0, 3 * pp // 16, dbody, 0)

        nech = ep // (16 * _TPB)

        def ebody(k, accs):
            e_acc, c_acc = accs
            j = t * nech + k
            ea = ev[pl.ds(j * 16, 16)]
            eb = ev[pl.ds(ep + j * 16, 16)]
            dd = zero
            diffs = []
            for c in range(3):
                va = plsc.load_gather(pv, [ea + c * pp])
                vb = plsc.load_gather(pv, [eb + c * pp])
                df = va - vb
                diffs.append(df)
                dd = dd + df * df
            e_acc = e_acc + dd
            gi = plsc.load_gather(iv, [ea])
            nn2 = zero
            dot = zero
            for c in range(3):
                nc = plsc.load_gather(nv, [gi + c * _NGT])
                nn2 = nn2 + nc * nc
                dot = dot + diffs[c] * nc
            cos = (jnp.abs(dot)
                   * _rsqrt_nr(jnp.maximum(dd, 1e-24))
                   * _rsqrt_nr(jnp.maximum(nn2, 1e-24)))
            return (e_acc, c_acc + cos)

        e_acc, c_acc = lax.fori_loop(0, nech, ebody, (zero, zero))

        nvch = pp // (16 * _TPB)

        def vbody(k, accs):
            l_acc, m_acc = accs
            j = t * nvch + k
            dcs = [dv[pl.ds(c * pp + j * 16, 16)] for c in range(3)]
            mm = dcs[0] * dcs[0] + dcs[1] * dcs[1] + dcs[2] * dcs[2]
            ns = [zero, zero, zero]
            for nb in range(8):
                idx = lv[pl.ds(nb * pp + j * 16, 16)]
                for c in range(3):
                    ns[c] = ns[c] + plsc.load_gather(dv, [idx + c * pp])
            cnt = lv[pl.ds(9 * pp + j * 16, 16)].astype(jnp.float32)
            ll = zero
            for c in range(3):
                lc = dcs[c] - ns[c] / cnt
                ll = ll + lc * lc
            return (l_acc + ll, m_acc + mm)

        l_acc, m_acc = lax.fori_loop(0, nvch, vbody, (zero, zero))

        ov[pl.ds((lvl * 4 + 0) * 16, 16)] = e_acc
        ov[pl.ds((lvl * 4 + 1) * 16, 16)] = c_acc
        ov[pl.ds((lvl * 4 + 2) * 16, 16)] = l_acc
        ov[pl.ds((lvl * 4 + 3) * 16, 16)] = m_acc

    pltpu.sync_copy(ov, out_hbm.at[wid])


def _sc_regularizers(predTs, beforeTs, edge_flats, lap_flats, idx2s,
                     normalsT, pps, eps):
    nlev = len(pps)
    scratch = []
    for i in range(nlev):
        scratch += [
            pltpu.VMEM((3 * pps[i],), jnp.float32),
            pltpu.VMEM((3 * pps[i],), jnp.float32),
            pltpu.VMEM((3 * pps[i],), jnp.float32),
            pltpu.VMEM((10 * pps[i],), jnp.int32),
            pltpu.VMEM((2 * eps[i],), jnp.int32),
            pltpu.VMEM((pps[i],), jnp.int32),
        ]
    scratch += [
        pltpu.VMEM((3 * _NGT,), jnp.float32),
        pltpu.VMEM((nlev * 4 * 16,), jnp.float32),
    ]
    fn = pl.kernel(
        functools.partial(_sc_body, tuple(pps), tuple(eps)),
        out_type=jax.ShapeDtypeStruct((_NW, nlev * 4 * 16), jnp.float32),
        mesh=_sc_mesh(),
        scratch_types=scratch,
        compiler_params=pltpu.CompilerParams(needs_layout_passes=False),
    )
    args = []
    for i in range(nlev):
        args += [predTs[i].reshape(_B, 3 * pps[i]),
                 beforeTs[i].reshape(_B, 3 * pps[i]),
                 edge_flats[i], lap_flats[i], idx2s[i]]
    args.append(normalsT.reshape(_B, 3 * _NGT))
    return fn(*args).reshape(_NW, nlev, 4, 16)


def kernel(pred0, pred1, pred2, pred_before0, pred_before1, pred_before2,
           gt_points, gt_normals,
           edges0, edges1, edges2, lap_idx0, lap_idx1, lap_idx2):
    preds = [pred0, pred1, pred2]
    befores = [pred_before0, pred_before1, pred_before2]
    edges = [edges0, edges1, edges2]
    laps = [lap_idx0, lap_idx1, lap_idx2]

    psizes, nes, pps, eps = [], [], [], []
    predTs, beforeTs, edge_flats, lap_flats = [], [], [], []
    for i in range(3):
        p = preds[i].shape[1]
        ne = edges[i].shape[0]
        pp = _round_up(p, 128)
        ep = _round_up(ne, 128)
        psizes.append(p)
        nes.append(ne)
        pps.append(pp)
        eps.append(ep)
        predTs.append(jnp.pad(jnp.transpose(preds[i], (0, 2, 1)),
                              ((0, 0), (0, 0), (0, pp - p))))
        beforeTs.append(jnp.pad(jnp.transpose(befores[i], (0, 2, 1)),
                                ((0, 0), (0, 0), (0, pp - p))))
        edge_flats.append(
            jnp.pad(edges[i], ((0, ep - ne), (0, 0))).T.reshape(-1))
        lapT = jnp.pad(laps[i], ((0, pp - p), (0, 0)), constant_values=p).T
        lapT = lapT.at[9, p:].set(1)
        lap_flats.append(lapT.reshape(-1))

    offs = [0, pps[0], pps[0] + pps[1]]
    segs = tuple((offs[i], pps[i], psizes[i]) for i in range(3))
    predT_cat = jnp.concatenate(predTs, axis=2)
    outs = _chamfer_all(gt_points, predT_cat, segs)
    d1s, d2s, idx2_all = outs[:3], outs[3:6], outs[6]
    idx2s = [idx2_all[:, 0, offs[i]:offs[i] + pps[i]] for i in range(3)]
    chamfers = [d1s[i][0, 0] / (_B * _NGT)
                + _W_CHAMFER_OPP * d2s[i][0, 0] / (_B * psizes[i])
                for i in range(3)]

    normalsT = jnp.transpose(gt_normals, (0, 2, 1))
    parts = _sc_regularizers(predTs, beforeTs, edge_flats, lap_flats,
                             idx2s, normalsT, pps, eps)
    sums = jnp.sum(parts, axis=(0, 3))

    loss = jnp.float32(0.0)
    for i in range(3):
        p, ne = psizes[i], nes[i]
        loss = loss + chamfers[i]
        loss = loss + _W_NORMAL * sums[i, 1] / (_B * ne)
        loss = loss + _W_EDGE * sums[i, 0] / (_B * ne)
        loss = loss + _W_LAPLACE * _LAP_CONST[i] * sums[i, 2] / (_B * p)
        if i > 0:
            loss = loss + _W_MOVE * _LAP_CONST[i] * sums[i, 3] / (_B * p)
    return loss
